# Initial kernel scaffold; baseline (speedup 1.0000x reference)
import jax
import jax.numpy as jnp
from jax.experimental import pallas as pl


def kernel(x, edge_index, W1, b1, W2, b2, W3, b3, Wo, bo):
    raise NotImplementedError("write your pallas kernel here")


# speedup vs baseline: 7.3809x; 7.3809x over previous
import functools

import jax
import jax.numpy as jnp
from jax import lax
from jax.experimental import pallas as pl
from jax.experimental.pallas import tpu as pltpu
from jax.experimental.pallas import tpu_sc as plsc

N = 10000
NP = 10240
E = 320000
D_IN = 128
NHID = 256
NCLASS = 40
NCP = 64

NC = 2
NS = 16
EB = 80
ROWS_PER_TILE = NP // NS

f32 = jnp.float32
i32 = jnp.int32


def _sc_mesh():
    return plsc.VectorSubcoreMesh(
        core_axis_name="c", subcore_axis_name="s", num_cores=NC, num_subcores=NS
    )


def _make_deg_kernel():
    ept = E // (NC * NS)
    nb = ept // EB

    @functools.partial(
        pl.kernel,
        out_type=[
            jax.ShapeDtypeStruct((NP, 16), f32),
            jax.ShapeDtypeStruct((NP, 16), f32),
        ],
        mesh=_sc_mesh(),
        scratch_types=[
            pltpu.VMEM((EB,), i32),
            pltpu.VMEM((EB, 16), f32),
            pltpu.VMEM_SHARED((NP, 16), f32),
        ],
        compiler_params=pltpu.CompilerParams(use_tc_tiling_on_sc=False),
    )
    def deg_kernel(dst_hbm, ones_hbm, deg0_hbm, deg1_hbm, dst_v, ones_v, acc_sh):
        c = lax.axis_index("c")
        s = lax.axis_index("s")
        r0 = s * ROWS_PER_TILE
        pltpu.sync_copy(ones_hbm, acc_sh.at[pl.ds(r0, ROWS_PER_TILE)])
        pltpu.sync_copy(ones_hbm.at[pl.ds(0, EB)], ones_v)
        plsc.subcore_barrier()
        base = c * (E // NC) + s * ept

        def body(i):
            pltpu.sync_copy(dst_hbm.at[pl.ds(base + i * EB, EB)], dst_v)
            pltpu.sync_copy(ones_v, acc_sh.at[dst_v], add=True)

        pl.loop(0, nb)(body)
        plsc.subcore_barrier()

        @pl.when(c == 0)
        def _():
            pltpu.sync_copy(
                acc_sh.at[pl.ds(r0, ROWS_PER_TILE)],
                deg0_hbm.at[pl.ds(r0, ROWS_PER_TILE)],
            )

        @pl.when(c == 1)
        def _():
            pltpu.sync_copy(
                acc_sh.at[pl.ds(r0, ROWS_PER_TILE)],
                deg1_hbm.at[pl.ds(r0, ROWS_PER_TILE)],
            )

    return deg_kernel


def _make_agg_kernel(dh):
    ept = E // NS
    nb = ept // EB

    @functools.partial(
        pl.kernel,
        out_type=[
            jax.ShapeDtypeStruct((NP, dh), f32),
            jax.ShapeDtypeStruct((NP, dh), f32),
        ],
        mesh=_sc_mesh(),
        scratch_types=[
            pltpu.VMEM((EB,), i32),
            pltpu.VMEM((EB,), i32),
            pltpu.VMEM((EB, dh), f32),
            pltpu.VMEM_SHARED((NP, dh), f32),
            pltpu.SemaphoreType.DMA,
        ],
        compiler_params=pltpu.CompilerParams(use_tc_tiling_on_sc=False),
    )
    def agg_kernel(hp0, hp1, src_hbm, dst_hbm, out0, out1,
                   src_v, dst_v, rows_v, acc_sh, sem):
        c = lax.axis_index("c")
        s = lax.axis_index("s")
        r0 = s * ROWS_PER_TILE

        @pl.when(c == 0)
        def _():
            pltpu.sync_copy(hp0.at[pl.ds(r0, ROWS_PER_TILE)],
                            acc_sh.at[pl.ds(r0, ROWS_PER_TILE)])

        @pl.when(c == 1)
        def _():
            pltpu.sync_copy(hp1.at[pl.ds(r0, ROWS_PER_TILE)],
                            acc_sh.at[pl.ds(r0, ROWS_PER_TILE)])

        plsc.subcore_barrier()
        base = s * ept

        def body(i):
            pltpu.sync_copy(src_hbm.at[pl.ds(base + i * EB, EB)], src_v)
            pltpu.sync_copy(dst_hbm.at[pl.ds(base + i * EB, EB)], dst_v)

            @pl.when(c == 0)
            def _():
                pltpu.async_copy(hp0.at[src_v], rows_v, sem).wait()

            @pl.when(c == 1)
            def _():
                pltpu.async_copy(hp1.at[src_v], rows_v, sem).wait()

            pltpu.sync_copy(rows_v, acc_sh.at[dst_v], add=True)

        pl.loop(0, nb)(body)
        plsc.subcore_barrier()

        @pl.when(c == 0)
        def _():
            pltpu.sync_copy(acc_sh.at[pl.ds(r0, ROWS_PER_TILE)],
                            out0.at[pl.ds(r0, ROWS_PER_TILE)])

        @pl.when(c == 1)
        def _():
            pltpu.sync_copy(acc_sh.at[pl.ds(r0, ROWS_PER_TILE)],
                            out1.at[pl.ds(r0, ROWS_PER_TILE)])

    return agg_kernel


_R = 512
_NBLK = NP // _R


def _prep_body(deg0_ref, deg1_ref, x_ref, dis_ref, hp0_ref, hp1_ref):
    d = deg0_ref[:, 0:1] + deg1_ref[:, 0:1] - 1.0
    dis = lax.rsqrt(d)
    dis_ref[...] = dis
    hp0_ref[...] = x_ref[:, : D_IN // 2] * dis
    hp1_ref[...] = x_ref[:, D_IN // 2 :] * dis


def _tc_prep(deg0, deg1, x_pad):
    return pl.pallas_call(
        _prep_body,
        grid=(_NBLK,),
        in_specs=[
            pl.BlockSpec((_R, 16), lambda i: (i, 0)),
            pl.BlockSpec((_R, 16), lambda i: (i, 0)),
            pl.BlockSpec((_R, D_IN), lambda i: (i, 0)),
        ],
        out_specs=[
            pl.BlockSpec((_R, 1), lambda i: (i, 0)),
            pl.BlockSpec((_R, D_IN // 2), lambda i: (i, 0)),
            pl.BlockSpec((_R, D_IN // 2), lambda i: (i, 0)),
        ],
        out_shape=[
            jax.ShapeDtypeStruct((NP, 1), f32),
            jax.ShapeDtypeStruct((NP, D_IN // 2), f32),
            jax.ShapeDtypeStruct((NP, D_IN // 2), f32),
        ],
    )(deg0, deg1, x_pad)


def _layer_body(acc0_ref, acc1_ref, dis_ref, w_ref, b_ref,
                out0_ref, out1_ref, *, dout):
    dis = dis_ref[...]
    z = jnp.concatenate([acc0_ref[...], acc1_ref[...]], axis=1) * dis
    h = jnp.dot(z, w_ref[...], preferred_element_type=f32) + b_ref[...]
    h = jnp.maximum(h, 0.0) * dis
    out0_ref[...] = h[:, : dout // 2]
    out1_ref[...] = h[:, dout // 2 :]


def _tc_layer(acc0, acc1, dis, w, b2d):
    din, dout = w.shape
    dh = acc0.shape[1]
    return pl.pallas_call(
        functools.partial(_layer_body, dout=dout),
        grid=(_NBLK,),
        in_specs=[
            pl.BlockSpec((_R, dh), lambda i: (i, 0)),
            pl.BlockSpec((_R, dh), lambda i: (i, 0)),
            pl.BlockSpec((_R, 1), lambda i: (i, 0)),
            pl.BlockSpec((din, dout), lambda i: (0, 0)),
            pl.BlockSpec((1, dout), lambda i: (0, 0)),
        ],
        out_specs=[
            pl.BlockSpec((_R, dout // 2), lambda i: (i, 0)),
            pl.BlockSpec((_R, dout // 2), lambda i: (i, 0)),
        ],
        out_shape=[
            jax.ShapeDtypeStruct((NP, dout // 2), f32),
            jax.ShapeDtypeStruct((NP, dout // 2), f32),
        ],
    )(acc0, acc1, dis, w, b2d)


def _outmm_body(h0_ref, h1_ref, wo_ref, z0_ref, z1_ref):
    a = jnp.concatenate([h0_ref[...], h1_ref[...]], axis=1)
    z = jnp.dot(a, wo_ref[...], preferred_element_type=f32)
    z0_ref[...] = z[:, : NCP // 2]
    z1_ref[...] = z[:, NCP // 2 :]


def _tc_outmm(hp0, hp1, wo_pad):
    return pl.pallas_call(
        _outmm_body,
        grid=(_NBLK,),
        in_specs=[
            pl.BlockSpec((_R, NHID // 2), lambda i: (i, 0)),
            pl.BlockSpec((_R, NHID // 2), lambda i: (i, 0)),
            pl.BlockSpec((NHID, NCP), lambda i: (0, 0)),
        ],
        out_specs=[
            pl.BlockSpec((_R, NCP // 2), lambda i: (i, 0)),
            pl.BlockSpec((_R, NCP // 2), lambda i: (i, 0)),
        ],
        out_shape=[
            jax.ShapeDtypeStruct((NP, NCP // 2), f32),
            jax.ShapeDtypeStruct((NP, NCP // 2), f32),
        ],
    )(hp0, hp1, wo_pad)


def _final_body(acc0_ref, acc1_ref, dis_ref, bo_ref, out_ref):
    t = jnp.concatenate([acc0_ref[...], acc1_ref[...]], axis=1)
    t = t * dis_ref[...] + bo_ref[...]
    col = lax.broadcasted_iota(i32, (_R, NCP), 1)
    valid = col < NCLASS
    tm = jnp.where(valid, t, -jnp.inf)
    m = jnp.max(tm, axis=1, keepdims=True)
    e = jnp.where(valid, jnp.exp(t - m), 0.0)
    lse = jnp.log(jnp.sum(e, axis=1, keepdims=True))
    out_ref[...] = t - m - lse


def _tc_final(acc0, acc1, dis, bo2d):
    return pl.pallas_call(
        _final_body,
        grid=(_NBLK,),
        in_specs=[
            pl.BlockSpec((_R, NCP // 2), lambda i: (i, 0)),
            pl.BlockSpec((_R, NCP // 2), lambda i: (i, 0)),
            pl.BlockSpec((_R, 1), lambda i: (i, 0)),
            pl.BlockSpec((1, NCP), lambda i: (0, 0)),
        ],
        out_specs=pl.BlockSpec((_R, NCP), lambda i: (i, 0)),
        out_shape=jax.ShapeDtypeStruct((NP, NCP), f32),
    )(acc0, acc1, dis, bo2d)


_sc_cache = {}


def _get_deg():
    if "deg" not in _sc_cache:
        _sc_cache["deg"] = _make_deg_kernel()
    return _sc_cache["deg"]


def _get_agg(dh):
    if dh not in _sc_cache:
        _sc_cache[dh] = _make_agg_kernel(dh)
    return _sc_cache[dh]


@jax.jit
def kernel(x, edge_index, W1, b1, W2, b2, W3, b3, Wo, bo):
    src = edge_index[0]
    dst = edge_index[1]
    x_pad = jnp.zeros((NP, D_IN), f32).at[:N].set(x)
    ones = jnp.ones((ROWS_PER_TILE, 16), f32)
    wo_pad = jnp.zeros((NHID, NCP), f32).at[:, :NCLASS].set(Wo)
    bo_pad = jnp.zeros((1, NCP), f32).at[0, :NCLASS].set(bo)

    deg0, deg1 = _get_deg()(dst, ones)
    dis, hp0a, hp0b = _tc_prep(deg0, deg1, x_pad)

    acc0a, acc0b = _get_agg(64)(hp0a, hp0b, src, dst)
    hp1a, hp1b = _tc_layer(acc0a, acc0b, dis, W1, b1.reshape(1, -1))

    acc1a, acc1b = _get_agg(128)(hp1a, hp1b, src, dst)
    hp2a, hp2b = _tc_layer(acc1a, acc1b, dis, W2, b2.reshape(1, -1))

    acc2a, acc2b = _get_agg(128)(hp2a, hp2b, src, dst)
    hp3a, hp3b = _tc_layer(acc2a, acc2b, dis, W3, b3.reshape(1, -1))

    zpa, zpb = _tc_outmm(hp3a, hp3b, wo_pad)
    acc3a, acc3b = _get_agg(32)(zpa, zpb, src, dst)
    out = _tc_final(acc3a, acc3b, dis, bo_pad)
    return out[:N, :NCLASS]


# speedup vs baseline: 9.0065x; 1.2203x over previous
import functools

import jax
import jax.numpy as jnp
from jax import lax
from jax.experimental import pallas as pl
from jax.experimental.pallas import tpu as pltpu
from jax.experimental.pallas import tpu_sc as plsc

N = 10000
NP = 10240
E = 320000
EP = 327680
D_IN = 128
NHID = 256
NCLASS = 40
NCP = 64

NC = 2
NS = 16
EB = 128
ROWS_PER_TILE = NP // NS

f32 = jnp.float32
i32 = jnp.int32


def _sc_mesh():
    return plsc.VectorSubcoreMesh(
        core_axis_name="c", subcore_axis_name="s", num_cores=NC, num_subcores=NS
    )


def _make_deg_kernel():
    ept = EP // (NC * NS)
    nb = ept // EB

    @functools.partial(
        pl.kernel,
        out_type=[
            jax.ShapeDtypeStruct((NP, 16), f32),
            jax.ShapeDtypeStruct((NP, 16), f32),
        ],
        mesh=_sc_mesh(),
        scratch_types=[
            pltpu.VMEM((EP // (NC * NS * EB), EB), i32),
            pltpu.VMEM((EB, 16), f32),
            pltpu.VMEM_SHARED((NP, 16), f32),
        ],
        compiler_params=pltpu.CompilerParams(use_tc_tiling_on_sc=False),
    )
    def deg_kernel(dst2d_hbm, ones_hbm, deg0_hbm, deg1_hbm, dst_v, ones_v, acc_sh):
        c = lax.axis_index("c")
        s = lax.axis_index("s")
        r0 = s * ROWS_PER_TILE
        pltpu.sync_copy(ones_hbm, acc_sh.at[pl.ds(r0, ROWS_PER_TILE)])
        pltpu.sync_copy(ones_hbm.at[pl.ds(0, EB)], ones_v)
        row0 = (c * NS + s) * nb
        pltpu.sync_copy(dst2d_hbm.at[pl.ds(row0, nb)], dst_v)
        plsc.subcore_barrier()

        def body(i):
            pltpu.sync_copy(ones_v, acc_sh.at[dst_v.at[i]], add=True)

        pl.loop(0, nb)(body)
        plsc.subcore_barrier()

        @pl.when(c == 0)
        def _():
            pltpu.sync_copy(
                acc_sh.at[pl.ds(r0, ROWS_PER_TILE)],
                deg0_hbm.at[pl.ds(r0, ROWS_PER_TILE)],
            )

        @pl.when(c == 1)
        def _():
            pltpu.sync_copy(
                acc_sh.at[pl.ds(r0, ROWS_PER_TILE)],
                deg1_hbm.at[pl.ds(r0, ROWS_PER_TILE)],
            )

    return deg_kernel


def _make_agg_kernel(dh):
    ept = EP // NS
    nb = ept // EB
    ch = 16 if dh > 64 else nb
    nch = nb // ch
    cpairs = ch // 2

    @functools.partial(
        pl.kernel,
        out_type=[
            jax.ShapeDtypeStruct((NP, dh), f32),
            jax.ShapeDtypeStruct((NP, dh), f32),
        ],
        mesh=_sc_mesh(),
        scratch_types=[
            pltpu.VMEM((ch, EB), i32),
            pltpu.VMEM((ch, EB), i32),
            pltpu.VMEM((EB, dh), f32),
            pltpu.VMEM((EB, dh), f32),
            pltpu.VMEM_SHARED((NP, dh), f32),
            pltpu.SemaphoreType.DMA,
            pltpu.SemaphoreType.DMA,
        ],
        compiler_params=pltpu.CompilerParams(use_tc_tiling_on_sc=False),
    )
    def agg_kernel(hp0, hp1, src2d, dst2d, out0, out1,
                   src_v, dst_v, buf_a, buf_b, acc_sh, sem_a, sem_b):
        c = lax.axis_index("c")
        s = lax.axis_index("s")
        r0 = s * ROWS_PER_TILE
        row0 = s * nb

        def run(tbl, out):
            rows = pl.ds(r0, ROWS_PER_TILE)
            pltpu.sync_copy(tbl.at[rows], acc_sh.at[rows])
            plsc.subcore_barrier()

            def chunk_body(ci):
                pltpu.sync_copy(src2d.at[pl.ds(row0 + ci * ch, ch)], src_v)
                pltpu.sync_copy(dst2d.at[pl.ds(row0 + ci * ch, ch)], dst_v)
                pltpu.async_copy(tbl.at[src_v.at[0]], buf_a, sem_a)

                def body(j):
                    i0 = 2 * j
                    pltpu.async_copy(tbl.at[src_v.at[i0 + 1]], buf_b, sem_b)
                    pltpu.make_async_copy(tbl.at[src_v.at[0]], buf_a, sem_a).wait()
                    pltpu.sync_copy(buf_a, acc_sh.at[dst_v.at[i0]], add=True)

                    @pl.when(j + 1 < cpairs)
                    def _():
                        pltpu.async_copy(tbl.at[src_v.at[i0 + 2]], buf_a, sem_a)

                    pltpu.make_async_copy(tbl.at[src_v.at[0]], buf_b, sem_b).wait()
                    pltpu.sync_copy(buf_b, acc_sh.at[dst_v.at[i0 + 1]], add=True)

                pl.loop(0, cpairs)(body)

            pl.loop(0, nch)(chunk_body)
            plsc.subcore_barrier()
            pltpu.sync_copy(acc_sh.at[rows], out.at[rows])

        @pl.when(c == 0)
        def _():
            run(hp0, out0)

        @pl.when(c == 1)
        def _():
            run(hp1, out1)

    return agg_kernel


_R = 512
_NBLK = NP // _R


def _prep_body(deg0_ref, deg1_ref, x_ref, dis_ref, hp0_ref, hp1_ref):
    d = deg0_ref[:, 0:1] + deg1_ref[:, 0:1] - 1.0
    dis = lax.rsqrt(d)
    dis_ref[...] = dis
    hp0_ref[...] = x_ref[:, : D_IN // 2] * dis
    hp1_ref[...] = x_ref[:, D_IN // 2 :] * dis


def _tc_prep(deg0, deg1, x_pad):
    return pl.pallas_call(
        _prep_body,
        grid=(_NBLK,),
        in_specs=[
            pl.BlockSpec((_R, 16), lambda i: (i, 0)),
            pl.BlockSpec((_R, 16), lambda i: (i, 0)),
            pl.BlockSpec((_R, D_IN), lambda i: (i, 0)),
        ],
        out_specs=[
            pl.BlockSpec((_R, 1), lambda i: (i, 0)),
            pl.BlockSpec((_R, D_IN // 2), lambda i: (i, 0)),
            pl.BlockSpec((_R, D_IN // 2), lambda i: (i, 0)),
        ],
        out_shape=[
            jax.ShapeDtypeStruct((NP, 1), f32),
            jax.ShapeDtypeStruct((NP, D_IN // 2), f32),
            jax.ShapeDtypeStruct((NP, D_IN // 2), f32),
        ],
    )(deg0, deg1, x_pad)


def _layer_body(acc0_ref, acc1_ref, dis_ref, w_ref, b_ref,
                out0_ref, out1_ref, *, dout):
    dis = dis_ref[...]
    z = jnp.concatenate([acc0_ref[...], acc1_ref[...]], axis=1) * dis
    h = jnp.dot(z, w_ref[...], preferred_element_type=f32) + b_ref[...]
    h = jnp.maximum(h, 0.0) * dis
    out0_ref[...] = h[:, : dout // 2]
    out1_ref[...] = h[:, dout // 2 :]


def _tc_layer(acc0, acc1, dis, w, b2d):
    din, dout = w.shape
    dh = acc0.shape[1]
    return pl.pallas_call(
        functools.partial(_layer_body, dout=dout),
        grid=(_NBLK,),
        in_specs=[
            pl.BlockSpec((_R, dh), lambda i: (i, 0)),
            pl.BlockSpec((_R, dh), lambda i: (i, 0)),
            pl.BlockSpec((_R, 1), lambda i: (i, 0)),
            pl.BlockSpec((din, dout), lambda i: (0, 0)),
            pl.BlockSpec((1, dout), lambda i: (0, 0)),
        ],
        out_specs=[
            pl.BlockSpec((_R, dout // 2), lambda i: (i, 0)),
            pl.BlockSpec((_R, dout // 2), lambda i: (i, 0)),
        ],
        out_shape=[
            jax.ShapeDtypeStruct((NP, dout // 2), f32),
            jax.ShapeDtypeStruct((NP, dout // 2), f32),
        ],
    )(acc0, acc1, dis, w, b2d)


def _outmm_body(h0_ref, h1_ref, wo_ref, z0_ref, z1_ref):
    a = jnp.concatenate([h0_ref[...], h1_ref[...]], axis=1)
    z = jnp.dot(a, wo_ref[...], preferred_element_type=f32)
    z0_ref[...] = z[:, : NCP // 2]
    z1_ref[...] = z[:, NCP // 2 :]


def _tc_outmm(hp0, hp1, wo_pad):
    return pl.pallas_call(
        _outmm_body,
        grid=(_NBLK,),
        in_specs=[
            pl.BlockSpec((_R, NHID // 2), lambda i: (i, 0)),
            pl.BlockSpec((_R, NHID // 2), lambda i: (i, 0)),
            pl.BlockSpec((NHID, NCP), lambda i: (0, 0)),
        ],
        out_specs=[
            pl.BlockSpec((_R, NCP // 2), lambda i: (i, 0)),
            pl.BlockSpec((_R, NCP // 2), lambda i: (i, 0)),
        ],
        out_shape=[
            jax.ShapeDtypeStruct((NP, NCP // 2), f32),
            jax.ShapeDtypeStruct((NP, NCP // 2), f32),
        ],
    )(hp0, hp1, wo_pad)


def _final_body(acc0_ref, acc1_ref, dis_ref, bo_ref, out_ref):
    t = jnp.concatenate([acc0_ref[...], acc1_ref[...]], axis=1)
    t = t * dis_ref[...] + bo_ref[...]
    col = lax.broadcasted_iota(i32, (_R, NCP), 1)
    valid = col < NCLASS
    tm = jnp.where(valid, t, -jnp.inf)
    m = jnp.max(tm, axis=1, keepdims=True)
    e = jnp.where(valid, jnp.exp(t - m), 0.0)
    lse = jnp.log(jnp.sum(e, axis=1, keepdims=True))
    out_ref[...] = t - m - lse


def _tc_final(acc0, acc1, dis, bo2d):
    return pl.pallas_call(
        _final_body,
        grid=(_NBLK,),
        in_specs=[
            pl.BlockSpec((_R, NCP // 2), lambda i: (i, 0)),
            pl.BlockSpec((_R, NCP // 2), lambda i: (i, 0)),
            pl.BlockSpec((_R, 1), lambda i: (i, 0)),
            pl.BlockSpec((1, NCP), lambda i: (0, 0)),
        ],
        out_specs=pl.BlockSpec((_R, NCP), lambda i: (i, 0)),
        out_shape=jax.ShapeDtypeStruct((NP, NCP), f32),
    )(acc0, acc1, dis, bo2d)


_sc_cache = {}


def _get_deg():
    if "deg" not in _sc_cache:
        _sc_cache["deg"] = _make_deg_kernel()
    return _sc_cache["deg"]


def _get_agg(dh):
    if dh not in _sc_cache:
        _sc_cache[dh] = _make_agg_kernel(dh)
    return _sc_cache[dh]


@jax.jit
def kernel(x, edge_index, W1, b1, W2, b2, W3, b3, Wo, bo):
    ei_pad = jnp.full((2, EP), NP - 1, i32).at[:, :E].set(edge_index)
    src = ei_pad[0].reshape(EP // EB, EB)
    dst = ei_pad[1].reshape(EP // EB, EB)
    x_pad = jnp.zeros((NP, D_IN), f32).at[:N].set(x)
    ones = jnp.ones((ROWS_PER_TILE, 16), f32)
    wo_pad = jnp.zeros((NHID, NCP), f32).at[:, :NCLASS].set(Wo)
    bo_pad = jnp.zeros((1, NCP), f32).at[0, :NCLASS].set(bo)

    deg0, deg1 = _get_deg()(dst, ones)
    dis, hp0a, hp0b = _tc_prep(deg0, deg1, x_pad)

    acc0a, acc0b = _get_agg(64)(hp0a, hp0b, src, dst)
    hp1a, hp1b = _tc_layer(acc0a, acc0b, dis, W1, b1.reshape(1, -1))

    acc1a, acc1b = _get_agg(128)(hp1a, hp1b, src, dst)
    hp2a, hp2b = _tc_layer(acc1a, acc1b, dis, W2, b2.reshape(1, -1))

    acc2a, acc2b = _get_agg(128)(hp2a, hp2b, src, dst)
    hp3a, hp3b = _tc_layer(acc2a, acc2b, dis, W3, b3.reshape(1, -1))

    zpa, zpb = _tc_outmm(hp3a, hp3b, wo_pad)
    acc3a, acc3b = _get_agg(32)(zpa, zpb, src, dst)
    out = _tc_final(acc3a, acc3b, dis, bo_pad)
    return out[:N, :NCLASS]
